# Initial kernel scaffold; baseline (speedup 1.0000x reference)
import functools

import jax
import jax.numpy as jnp
from jax import lax
from jax.experimental import pallas as pl
from jax.experimental.pallas import tpu as pltpu
from jax.experimental.pallas import tpu_sc as plsc

N_NODES = 10000
N_EDGES = 320000
F = 128
EMB = 64
NG = 256

NC = 2
NS = 16
NW = NC * NS

_EPB = N_EDGES // NW
_C = 80
_NCHUNK = _EPB // _C

_NPT = 320
_NPAD = _NPT * NW
_SEG = 264


def _bn(x, g, c):
    mu = jnp.mean(x, axis=0, keepdims=True)
    var = jnp.mean((x - mu) ** 2, axis=0, keepdims=True)
    return (x - mu) / jnp.sqrt(var + 1e-5) * g + c


def _bn0_body(x_ref, g_ref, c_ref, o_ref):
    o_ref[...] = _bn(x_ref[...], g_ref[...], c_ref[...])


def _bn0(x, g, c):
    return pl.pallas_call(
        _bn0_body,
        out_shape=jax.ShapeDtypeStruct(x.shape, jnp.float32),
    )(x, g.reshape(1, -1), c.reshape(1, -1))


def _edge_linear_body(ea_ref, we_ref, be_ref, o_ref):
    o_ref[...] = (
        jnp.dot(ea_ref[...], we_ref[...], preferred_element_type=jnp.float32)
        + be_ref[...]
    )


def _edge_linear(ea, we, be):
    blk = 16000
    grid = N_EDGES // blk
    return pl.pallas_call(
        _edge_linear_body,
        grid=(grid,),
        in_specs=[
            pl.BlockSpec((blk, 16), lambda i: (i, 0)),
            pl.BlockSpec((16, F), lambda i: (0, 0)),
            pl.BlockSpec((1, F), lambda i: (0, 0)),
        ],
        out_specs=pl.BlockSpec((blk, F), lambda i: (i, 0)),
        out_shape=jax.ShapeDtypeStruct((N_EDGES, F), jnp.float32),
    )(ea, we, be.reshape(1, -1))


def _node_mlp_body(relu_out, x_ref, agg_ref, w1_ref, b1_ref, g1_ref, c1_ref,
                   w2_ref, b2_ref, go_ref, co_ref, o_ref):
    h = x_ref[...] + agg_ref[0] + agg_ref[1]
    h = jnp.dot(h, w1_ref[...], preferred_element_type=jnp.float32) + b1_ref[...]
    h = jnp.maximum(_bn(h, g1_ref[...], c1_ref[...]), 0.0)
    h = jnp.dot(h, w2_ref[...], preferred_element_type=jnp.float32) + b2_ref[...]
    h = _bn(h, go_ref[...], co_ref[...])
    if relu_out:
        h = jnp.maximum(h, 0.0)
    o_ref[...] = h


def _node_mlp(x, agg, w1, b1, g1, c1, w2, b2, go, co, relu_out):
    out_d = w2.shape[1]
    return pl.pallas_call(
        functools.partial(_node_mlp_body, relu_out),
        out_shape=jax.ShapeDtypeStruct((N_NODES, out_d), jnp.float32),
    )(x, agg, w1, b1.reshape(1, -1), g1.reshape(1, -1), c1.reshape(1, -1),
      w2, b2.reshape(1, -1), go.reshape(1, -1), co.reshape(1, -1))


def _pool_combine_body(x_ref, b_ref, mp_ref, o_ref):
    x = x_ref[...]
    seg = lax.broadcasted_iota(jnp.int32, (N_NODES, NG), 1)
    oh = (b_ref[...] == seg).astype(jnp.float32)
    sums = lax.dot_general(oh, x, (((0,), (0,)), ((), ())),
                           preferred_element_type=jnp.float32)
    cnt = jnp.sum(oh, axis=0)[:, None]
    mean = sums / jnp.maximum(cnt, 1.0)
    mx = jnp.max(mp_ref[...], axis=0)[:NG, :]
    mx = jnp.where(cnt > 0.0, mx, 0.0)
    o_ref[...] = jnp.concatenate([mean, mx], axis=1)


def _pool_combine(x, batch, maxpart):
    return pl.pallas_call(
        _pool_combine_body,
        out_shape=jax.ShapeDtypeStruct((NG, 2 * EMB), jnp.float32),
    )(x, batch.reshape(N_NODES, 1), maxpart)


def _classifier_body(e1_ref, e2_ref, gf1_ref, gf2_ref, fp1_ref, fp2_ref,
                     wc1_ref, bc1_ref, gc1_ref, cc1_ref,
                     wc2_ref, bc2_ref, gc2_ref, cc2_ref,
                     wc3_ref, bc3_ref, o_ref):
    e1 = e1_ref[...]
    e2 = e2_ref[...]
    comb = jnp.concatenate(
        [e1, e2, jnp.abs(e1 - e2), e1 * e2,
         gf1_ref[...], gf2_ref[...], fp1_ref[...], fp2_ref[...]], axis=1)
    h = jnp.dot(comb, wc1_ref[...], preferred_element_type=jnp.float32) + bc1_ref[...]
    h = jnp.maximum(_bn(h, gc1_ref[...], cc1_ref[...]), 0.0)
    h = jnp.dot(h, wc2_ref[...], preferred_element_type=jnp.float32) + bc2_ref[...]
    h = jnp.maximum(_bn(h, gc2_ref[...], cc2_ref[...]), 0.0)
    o_ref[...] = (
        jnp.dot(h, wc3_ref[...], preferred_element_type=jnp.float32) + bc3_ref[...]
    )


def _classifier(e1, e2, gf1, gf2, fp1, fp2, wc1, bc1, gc1, cc1,
                wc2, bc2, gc2, cc2, wc3, bc3):
    return pl.pallas_call(
        _classifier_body,
        out_shape=jax.ShapeDtypeStruct((NG, 1), jnp.float32),
    )(e1, e2, gf1, gf2, fp1, fp2,
      wc1, bc1.reshape(1, -1), gc1.reshape(1, -1), cc1.reshape(1, -1),
      wc2, bc2.reshape(1, -1), gc2.reshape(1, -1), cc2.reshape(1, -1),
      wc3, bc3.reshape(1, -1))


_SC_MESH = plsc.VectorSubcoreMesh(
    core_axis_name="c", subcore_axis_name="s", num_cores=NC, num_subcores=NS)


def _sc_edge_body(x_hbm, e_hbm, src_hbm, dst_hbm, z_hbm, out_hbm,
                  srcv, dstv, xv, ev, agg, sem):
    cid = lax.axis_index("c")
    sid = lax.axis_index("s")
    wid = cid * NS + sid
    rows_per_tile = N_NODES // NS

    pltpu.sync_copy(z_hbm, agg.at[pl.ds(sid * rows_per_tile, rows_per_tile)])
    plsc.subcore_barrier()

    tbase = wid * _EPB

    def chunk(k, _):
        base = tbase + k * _C
        pltpu.sync_copy(src_hbm.at[pl.ds(base, _C)], srcv)
        pltpu.sync_copy(dst_hbm.at[pl.ds(base, _C)], dstv)
        pltpu.sync_copy(e_hbm.at[pl.ds(base, _C)], ev)
        pltpu.async_copy(x_hbm.at[srcv], xv, sem).wait()

        def row(r, _):
            for j in range(F // 16):
                s = pl.ds(16 * j, 16)
                xv[r, s] = jnp.maximum(xv[r, s] + ev[r, s], 0.0)
            return 0

        lax.fori_loop(0, _C, row, 0)
        pltpu.sync_copy(xv, agg.at[dstv], add=True)
        return 0

    lax.fori_loop(0, _NCHUNK, chunk, 0)
    plsc.subcore_barrier()
    pltpu.sync_copy(agg.at[pl.ds(sid * rows_per_tile, rows_per_tile)],
                    out_hbm.at[cid, pl.ds(sid * rows_per_tile, rows_per_tile)])


@functools.partial(
    pl.kernel,
    out_type=jax.ShapeDtypeStruct((NC, N_NODES, F), jnp.float32),
    mesh=_SC_MESH,
    scratch_types=[
        pltpu.VMEM((_C,), jnp.int32),
        pltpu.VMEM((_C,), jnp.int32),
        pltpu.VMEM((_C, F), jnp.float32),
        pltpu.VMEM((_C, F), jnp.float32),
        pltpu.VMEM_SHARED((N_NODES, F), jnp.float32),
        pltpu.SemaphoreType.DMA,
    ],
)
def _sc_edge(x_hbm, e_hbm, src_hbm, dst_hbm, z_hbm, out_hbm,
             srcv, dstv, xv, ev, agg, sem):
    _sc_edge_body(x_hbm, e_hbm, src_hbm, dst_hbm, z_hbm, out_hbm,
                  srcv, dstv, xv, ev, agg, sem)


def _sc_maxpool_body(x_hbm, b_hbm, out_hbm, idv, xv, acc):
    cid = lax.axis_index("c")
    sid = lax.axis_index("s")
    wid = cid * NS + sid
    base = wid * _NPT

    neg = jnp.full((16,), -1e30, jnp.float32)

    def zrow(r, _):
        for j in range(EMB // 16):
            acc[r, pl.ds(16 * j, 16)] = neg
        return 0

    lax.fori_loop(0, _SEG, zrow, 0)

    pltpu.sync_copy(b_hbm.at[pl.ds(base, _NPT)], idv)
    pltpu.sync_copy(x_hbm.at[pl.ds(base, _NPT)], xv)

    def row(r, _):
        seg = idv[r]
        for j in range(EMB // 16):
            s = pl.ds(16 * j, 16)
            acc[seg, s] = jnp.maximum(acc[seg, s], xv[r, s])
        return 0

    lax.fori_loop(0, _NPT, row, 0)
    pltpu.sync_copy(acc, out_hbm.at[wid])


@functools.partial(
    pl.kernel,
    out_type=jax.ShapeDtypeStruct((NW, _SEG, EMB), jnp.float32),
    mesh=_SC_MESH,
    scratch_types=[
        pltpu.VMEM((_NPT,), jnp.int32),
        pltpu.VMEM((_NPT, EMB), jnp.float32),
        pltpu.VMEM((_SEG, EMB), jnp.float32),
    ],
)
def _sc_maxpool(x_hbm, b_hbm, out_hbm, idv, xv, acc):
    _sc_maxpool_body(x_hbm, b_hbm, out_hbm, idv, xv, acc)


def _encoder(x, ei, ea, batch, p, pre_w):
    src = ei[0]
    dst = ei[1]
    zeros = jnp.zeros((N_NODES // NS, F), jnp.float32)

    h = _bn0(x, p["g0"], p["c0"])
    for li, (pre, relu_out) in enumerate(
            [("l1_", True), ("l2_", True), ("l3_", False)]):
        w = {k: pre_w[pre + k] for k in
             ("We", "be", "W1", "b1", "g1", "c1", "W2", "b2")}
        e = _edge_linear(ea, w["We"], w["be"])
        agg = _sc_edge(h, e, src, dst, zeros)
        go, co = [("g1", "c1"), ("g2", "c2"), ("g3", "c3")][li]
        h = _node_mlp(h, agg, w["W1"], w["b1"], w["g1"], w["c1"],
                      w["W2"], w["b2"], p[go], p[co], relu_out)

    xp = jnp.pad(h, ((0, _NPAD - N_NODES), (0, 0)))
    bp = jnp.pad(batch, (0, _NPAD - N_NODES), constant_values=NG)
    maxpart = _sc_maxpool(xp, bp)
    return _pool_combine(h, batch, maxpart)


def kernel(x1, edge_index1, edge_attr1, batch1, gf1, fp1,
           x2, edge_index2, edge_attr2, batch2, gf2, fp2,
           g0, c0, g1, c1, g2, c2, g3, c3,
           l1_We, l1_be, l1_W1, l1_b1, l1_g1, l1_c1, l1_W2, l1_b2,
           l2_We, l2_be, l2_W1, l2_b1, l2_g1, l2_c1, l2_W2, l2_b2,
           l3_We, l3_be, l3_W1, l3_b1, l3_g1, l3_c1, l3_W2, l3_b2,
           Wc1, bc1, gc1, cc1, Wc2, bc2, gc2, cc2, Wc3, bc3):
    p = dict(g0=g0, c0=c0, g1=g1, c1=c1, g2=g2, c2=c2, g3=g3, c3=c3)
    pre_w = {}
    for pre, vals in [
        ("l1_", (l1_We, l1_be, l1_W1, l1_b1, l1_g1, l1_c1, l1_W2, l1_b2)),
        ("l2_", (l2_We, l2_be, l2_W1, l2_b1, l2_g1, l2_c1, l2_W2, l2_b2)),
        ("l3_", (l3_We, l3_be, l3_W1, l3_b1, l3_g1, l3_c1, l3_W2, l3_b2)),
    ]:
        for k, v in zip(("We", "be", "W1", "b1", "g1", "c1", "W2", "b2"), vals):
            pre_w[pre + k] = v

    p1 = _encoder(x1, edge_index1, edge_attr1, batch1, p, pre_w)
    p2 = _encoder(x2, edge_index2, edge_attr2, batch2, p, pre_w)
    return _classifier(p1, p2, gf1, gf2, fp1, fp2,
                       Wc1, bc1, gc1, cc1, Wc2, bc2, gc2, cc2, Wc3, bc3)


# speedup vs baseline: 2.1016x; 2.1016x over previous
import functools

import jax
import jax.numpy as jnp
from jax import lax
from jax.experimental import pallas as pl
from jax.experimental.pallas import tpu as pltpu
from jax.experimental.pallas import tpu_sc as plsc

N_NODES = 10000
N_EDGES = 320000
F = 128
EMB = 64
NG = 256

NC = 2
NS = 16
NW = NC * NS

_EPB = N_EDGES // NW
_C = 80
_NCHUNK = _EPB // _C

_NROWS = 10240
_NPT = 320
_NPAD = _NPT * NW
_SEG = 264


def _bn(x, g, c):
    mu = jnp.mean(x, axis=0, keepdims=True)
    var = jnp.mean((x - mu) ** 2, axis=0, keepdims=True)
    return (x - mu) / jnp.sqrt(var + 1e-5) * g + c


def _bn0_body(x_ref, g_ref, c_ref, o_ref):
    o_ref[...] = _bn(x_ref[...], g_ref[...], c_ref[...])


def _bn0(x, g, c):
    return pl.pallas_call(
        _bn0_body,
        out_shape=jax.ShapeDtypeStruct(x.shape, jnp.float32),
    )(x, g.reshape(1, -1), c.reshape(1, -1))


def _edge_linear_body(ea_ref, we_ref, be_ref, o_ref):
    o_ref[...] = (
        jnp.dot(ea_ref[...], we_ref[...], preferred_element_type=jnp.float32)
        + be_ref[...]
    )


def _edge_linear(ea, we, be):
    blk = 8000
    grid = N_EDGES // blk
    return pl.pallas_call(
        _edge_linear_body,
        grid=(grid,),
        in_specs=[
            pl.BlockSpec((blk, 16), lambda i: (i, 0)),
            pl.BlockSpec((16, F), lambda i: (0, 0)),
            pl.BlockSpec((1, F), lambda i: (0, 0)),
        ],
        out_specs=pl.BlockSpec((blk, F), lambda i: (i, 0)),
        out_shape=jax.ShapeDtypeStruct((N_EDGES, F), jnp.float32),
    )(ea, we, be.reshape(1, -1))


def _node_mlp_body(relu_out, x_ref, agg_ref, w1_ref, b1_ref, g1_ref, c1_ref,
                   w2_ref, b2_ref, go_ref, co_ref, o_ref):
    h = x_ref[...] + agg_ref[0, :N_NODES] + agg_ref[1, :N_NODES]
    h = jnp.dot(h, w1_ref[...], preferred_element_type=jnp.float32) + b1_ref[...]
    h = jnp.maximum(_bn(h, g1_ref[...], c1_ref[...]), 0.0)
    h = jnp.dot(h, w2_ref[...], preferred_element_type=jnp.float32) + b2_ref[...]
    h = _bn(h, go_ref[...], co_ref[...])
    if relu_out:
        h = jnp.maximum(h, 0.0)
    o_ref[...] = h


def _node_mlp(x, agg, w1, b1, g1, c1, w2, b2, go, co, relu_out):
    out_d = w2.shape[1]
    return pl.pallas_call(
        functools.partial(_node_mlp_body, relu_out),
        out_shape=jax.ShapeDtypeStruct((N_NODES, out_d), jnp.float32),
    )(x, agg, w1, b1.reshape(1, -1), g1.reshape(1, -1), c1.reshape(1, -1),
      w2, b2.reshape(1, -1), go.reshape(1, -1), co.reshape(1, -1))


def _pool_combine_body(x_ref, b_ref, mp_ref, o_ref):
    x = x_ref[...]
    seg = lax.broadcasted_iota(jnp.int32, (N_NODES, NG), 1)
    oh = (b_ref[...] == seg).astype(jnp.float32)
    sums = lax.dot_general(oh, x, (((0,), (0,)), ((), ())),
                           preferred_element_type=jnp.float32,
                           precision=lax.Precision.HIGHEST)
    cnt = jnp.sum(oh, axis=0)[:, None]
    mean = sums / jnp.maximum(cnt, 1.0)
    mx = jnp.max(mp_ref[...], axis=0)[:NG, :]
    mx = jnp.where(cnt > 0.0, mx, 0.0)
    o_ref[...] = jnp.concatenate([mean, mx], axis=1)


def _pool_combine(x, batch, maxpart):
    return pl.pallas_call(
        _pool_combine_body,
        out_shape=jax.ShapeDtypeStruct((NG, 2 * EMB), jnp.float32),
    )(x, batch.reshape(N_NODES, 1), maxpart)


def _classifier_body(e1_ref, e2_ref, gf1_ref, gf2_ref, fp1_ref, fp2_ref,
                     wc1_ref, bc1_ref, gc1_ref, cc1_ref,
                     wc2_ref, bc2_ref, gc2_ref, cc2_ref,
                     wc3_ref, bc3_ref, o_ref):
    e1 = e1_ref[...]
    e2 = e2_ref[...]
    comb = jnp.concatenate(
        [e1, e2, jnp.abs(e1 - e2), e1 * e2,
         gf1_ref[...], gf2_ref[...], fp1_ref[...], fp2_ref[...]], axis=1)
    h = jnp.dot(comb, wc1_ref[...], preferred_element_type=jnp.float32) + bc1_ref[...]
    h = jnp.maximum(_bn(h, gc1_ref[...], cc1_ref[...]), 0.0)
    h = jnp.dot(h, wc2_ref[...], preferred_element_type=jnp.float32) + bc2_ref[...]
    h = jnp.maximum(_bn(h, gc2_ref[...], cc2_ref[...]), 0.0)
    o_ref[...] = (
        jnp.dot(h, wc3_ref[...], preferred_element_type=jnp.float32) + bc3_ref[...]
    )


def _classifier(e1, e2, gf1, gf2, fp1, fp2, wc1, bc1, gc1, cc1,
                wc2, bc2, gc2, cc2, wc3, bc3):
    return pl.pallas_call(
        _classifier_body,
        out_shape=jax.ShapeDtypeStruct((NG, 1), jnp.float32),
    )(e1, e2, gf1, gf2, fp1, fp2,
      wc1, bc1.reshape(1, -1), gc1.reshape(1, -1), cc1.reshape(1, -1),
      wc2, bc2.reshape(1, -1), gc2.reshape(1, -1), cc2.reshape(1, -1),
      wc3, bc3.reshape(1, -1))


_SC_MESH = plsc.VectorSubcoreMesh(
    core_axis_name="c", subcore_axis_name="s", num_cores=NC, num_subcores=NS)


def _sc_edge_body(x_hbm, e_hbm, src_hbm, dst_hbm, z_hbm, out_hbm,
                  srcv, dstv, xv, ev, agg, sem):
    cid = lax.axis_index("c")
    sid = lax.axis_index("s")
    wid = cid * NS + sid
    rows_per_tile = _NROWS // NS

    pltpu.sync_copy(z_hbm, agg.at[pl.ds(sid * rows_per_tile, rows_per_tile)])
    plsc.subcore_barrier()

    tbase = wid * _EPB

    def chunk(k, _):
        base = tbase + k * _C
        pltpu.sync_copy(src_hbm.at[pl.ds(base, _C)], srcv)
        pltpu.sync_copy(dst_hbm.at[pl.ds(base, _C)], dstv)
        pltpu.sync_copy(e_hbm.at[pl.ds(base, _C)], ev)
        pltpu.async_copy(x_hbm.at[srcv], xv, sem).wait()

        def row(r, _):
            for j in range(F // 16):
                s = pl.ds(16 * j, 16)
                xv[r, s] = jnp.maximum(xv[r, s] + ev[r, s], 0.0)
            return 0

        lax.fori_loop(0, _C, row, 0)
        pltpu.sync_copy(xv, agg.at[dstv], add=True)
        return 0

    lax.fori_loop(0, _NCHUNK, chunk, 0)
    plsc.subcore_barrier()
    pltpu.sync_copy(agg.at[pl.ds(sid * rows_per_tile, rows_per_tile)],
                    out_hbm.at[cid, pl.ds(sid * rows_per_tile, rows_per_tile)])


@functools.partial(
    pl.kernel,
    out_type=jax.ShapeDtypeStruct((NC, _NROWS, F), jnp.float32),
    mesh=_SC_MESH,
    scratch_types=[
        pltpu.VMEM((_C,), jnp.int32),
        pltpu.VMEM((_C,), jnp.int32),
        pltpu.VMEM((_C, F), jnp.float32),
        pltpu.VMEM((_C, F), jnp.float32),
        pltpu.VMEM_SHARED((_NROWS, F), jnp.float32),
        pltpu.SemaphoreType.DMA,
    ],
)
def _sc_edge(x_hbm, e_hbm, src_hbm, dst_hbm, z_hbm, out_hbm,
             srcv, dstv, xv, ev, agg, sem):
    _sc_edge_body(x_hbm, e_hbm, src_hbm, dst_hbm, z_hbm, out_hbm,
                  srcv, dstv, xv, ev, agg, sem)


def _sc_maxpool_body(x_hbm, b_hbm, out_hbm, idv, xv, acc):
    cid = lax.axis_index("c")
    sid = lax.axis_index("s")
    wid = cid * NS + sid
    base = wid * _NPT

    neg = jnp.full((16,), -1e30, jnp.float32)

    def zrow(r, _):
        for j in range(EMB // 16):
            acc[r, pl.ds(16 * j, 16)] = neg
        return 0

    lax.fori_loop(0, _SEG, zrow, 0)

    pltpu.sync_copy(b_hbm.at[pl.ds(base, _NPT)], idv.at[pl.ds(0, _NPT)])
    pltpu.sync_copy(x_hbm.at[pl.ds(base, _NPT)], xv)

    def row(r, _):
        seg = idv[pl.ds(r, 16)][0]
        for j in range(EMB // 16):
            s = pl.ds(16 * j, 16)
            acc[seg, s] = jnp.maximum(acc[seg, s], xv[r, s])
        return 0

    lax.fori_loop(0, _NPT, row, 0)
    pltpu.sync_copy(acc, out_hbm.at[wid])


@functools.partial(
    pl.kernel,
    out_type=jax.ShapeDtypeStruct((NW, _SEG, EMB), jnp.float32),
    mesh=_SC_MESH,
    scratch_types=[
        pltpu.VMEM((_NPT + 16,), jnp.int32),
        pltpu.VMEM((_NPT, EMB), jnp.float32),
        pltpu.VMEM((_SEG, EMB), jnp.float32),
    ],
)
def _sc_maxpool(x_hbm, b_hbm, out_hbm, idv, xv, acc):
    _sc_maxpool_body(x_hbm, b_hbm, out_hbm, idv, xv, acc)


def _encoder(x, ei, ea, batch, p, pre_w):
    src = ei[0]
    dst = ei[1]
    zeros = jnp.zeros((_NROWS // NS, F), jnp.float32)

    h = _bn0(x, p["g0"], p["c0"])
    for li, (pre, relu_out) in enumerate(
            [("l1_", True), ("l2_", True), ("l3_", False)]):
        w = {k: pre_w[pre + k] for k in
             ("We", "be", "W1", "b1", "g1", "c1", "W2", "b2")}
        e = _edge_linear(ea, w["We"], w["be"])
        agg = _sc_edge(h, e, src, dst, zeros)
        go, co = [("g1", "c1"), ("g2", "c2"), ("g3", "c3")][li]
        h = _node_mlp(h, agg, w["W1"], w["b1"], w["g1"], w["c1"],
                      w["W2"], w["b2"], p[go], p[co], relu_out)

    xp = jnp.pad(h, ((0, _NPAD - N_NODES), (0, 0)))
    bp = jnp.pad(batch, (0, _NPAD - N_NODES), constant_values=NG)
    maxpart = _sc_maxpool(xp, bp)
    return _pool_combine(h, batch, maxpart)


def kernel(x1, edge_index1, edge_attr1, batch1, gf1, fp1,
           x2, edge_index2, edge_attr2, batch2, gf2, fp2,
           g0, c0, g1, c1, g2, c2, g3, c3,
           l1_We, l1_be, l1_W1, l1_b1, l1_g1, l1_c1, l1_W2, l1_b2,
           l2_We, l2_be, l2_W1, l2_b1, l2_g1, l2_c1, l2_W2, l2_b2,
           l3_We, l3_be, l3_W1, l3_b1, l3_g1, l3_c1, l3_W2, l3_b2,
           Wc1, bc1, gc1, cc1, Wc2, bc2, gc2, cc2, Wc3, bc3):
    p = dict(g0=g0, c0=c0, g1=g1, c1=c1, g2=g2, c2=c2, g3=g3, c3=c3)
    pre_w = {}
    for pre, vals in [
        ("l1_", (l1_We, l1_be, l1_W1, l1_b1, l1_g1, l1_c1, l1_W2, l1_b2)),
        ("l2_", (l2_We, l2_be, l2_W1, l2_b1, l2_g1, l2_c1, l2_W2, l2_b2)),
        ("l3_", (l3_We, l3_be, l3_W1, l3_b1, l3_g1, l3_c1, l3_W2, l3_b2)),
    ]:
        for k, v in zip(("We", "be", "W1", "b1", "g1", "c1", "W2", "b2"), vals):
            pre_w[pre + k] = v

    p1 = _encoder(x1, edge_index1, edge_attr1, batch1, p, pre_w)
    p2 = _encoder(x2, edge_index2, edge_attr2, batch2, p, pre_w)
    return _classifier(p1, p2, gf1, gf2, fp1, fp2,
                       Wc1, bc1, gc1, cc1, Wc2, bc2, gc2, cc2, Wc3, bc3)


# speedup vs baseline: 3.9237x; 1.8670x over previous
import functools

import jax
import jax.numpy as jnp
from jax import lax
from jax.experimental import pallas as pl
from jax.experimental.pallas import tpu as pltpu
from jax.experimental.pallas import tpu_sc as plsc

N_NODES = 10000
N_EDGES = 320000
F = 128
EMB = 64
NG = 256

NC = 2
NS = 16
NW = NC * NS

_EPB = N_EDGES // NW
_C = 80
_NCHUNK = _EPB // _C

_NROWS = 10240
_NPT = 320
_NPAD = _NPT * NW
_SEG = 264


def _bn(x, g, c):
    mu = jnp.mean(x, axis=0, keepdims=True)
    var = jnp.mean((x - mu) ** 2, axis=0, keepdims=True)
    return (x - mu) / jnp.sqrt(var + 1e-5) * g + c


def _bn0_body(x_ref, g_ref, c_ref, o_ref):
    o_ref[...] = _bn(x_ref[...], g_ref[...], c_ref[...])


def _bn0(x, g, c):
    return pl.pallas_call(
        _bn0_body,
        out_shape=jax.ShapeDtypeStruct(x.shape, jnp.float32),
    )(x, g.reshape(1, -1), c.reshape(1, -1))


def _edge_linear_body(ea_ref, we_ref, be_ref, o_ref):
    o_ref[...] = (
        jnp.dot(ea_ref[...], we_ref[...], preferred_element_type=jnp.float32)
        + be_ref[...]
    )


def _edge_linear(ea, we, be):
    blk = 8000
    grid = N_EDGES // blk
    return pl.pallas_call(
        _edge_linear_body,
        grid=(grid,),
        in_specs=[
            pl.BlockSpec((blk, 16), lambda i: (i, 0)),
            pl.BlockSpec((16, F), lambda i: (0, 0)),
            pl.BlockSpec((1, F), lambda i: (0, 0)),
        ],
        out_specs=pl.BlockSpec((blk, F), lambda i: (i, 0)),
        out_shape=jax.ShapeDtypeStruct((N_EDGES, F), jnp.float32),
    )(ea, we, be.reshape(1, -1))


def _node_mlp_body(relu_out, x_ref, agg_ref, w1_ref, b1_ref, g1_ref, c1_ref,
                   w2_ref, b2_ref, go_ref, co_ref, o_ref):
    h = x_ref[...] + agg_ref[0, :N_NODES] + agg_ref[1, :N_NODES]
    h = jnp.dot(h, w1_ref[...], preferred_element_type=jnp.float32) + b1_ref[...]
    h = jnp.maximum(_bn(h, g1_ref[...], c1_ref[...]), 0.0)
    h = jnp.dot(h, w2_ref[...], preferred_element_type=jnp.float32) + b2_ref[...]
    h = _bn(h, go_ref[...], co_ref[...])
    if relu_out:
        h = jnp.maximum(h, 0.0)
    o_ref[...] = h


def _node_mlp(x, agg, w1, b1, g1, c1, w2, b2, go, co, relu_out):
    out_d = w2.shape[1]
    return pl.pallas_call(
        functools.partial(_node_mlp_body, relu_out),
        out_shape=jax.ShapeDtypeStruct((N_NODES, out_d), jnp.float32),
    )(x, agg, w1, b1.reshape(1, -1), g1.reshape(1, -1), c1.reshape(1, -1),
      w2, b2.reshape(1, -1), go.reshape(1, -1), co.reshape(1, -1))


def _pool_combine_body(x_ref, b_ref, mp_ref, o_ref):
    x = x_ref[...]
    seg = lax.broadcasted_iota(jnp.int32, (N_NODES, NG), 1)
    oh = (b_ref[...] == seg).astype(jnp.float32)
    sums = lax.dot_general(oh, x, (((0,), (0,)), ((), ())),
                           preferred_element_type=jnp.float32,
                           precision=lax.Precision.HIGHEST)
    cnt = jnp.sum(oh, axis=0)[:, None]
    mean = sums / jnp.maximum(cnt, 1.0)
    mx = jnp.max(mp_ref[...], axis=0)[:NG, :]
    mx = jnp.where(cnt > 0.0, mx, 0.0)
    o_ref[...] = jnp.concatenate([mean, mx], axis=1)


def _pool_combine(x, batch, maxpart):
    return pl.pallas_call(
        _pool_combine_body,
        out_shape=jax.ShapeDtypeStruct((NG, 2 * EMB), jnp.float32),
    )(x, batch.reshape(N_NODES, 1), maxpart)


def _classifier_body(e1_ref, e2_ref, gf1_ref, gf2_ref, fp1_ref, fp2_ref,
                     wc1_ref, bc1_ref, gc1_ref, cc1_ref,
                     wc2_ref, bc2_ref, gc2_ref, cc2_ref,
                     wc3_ref, bc3_ref, o_ref):
    e1 = e1_ref[...]
    e2 = e2_ref[...]
    comb = jnp.concatenate(
        [e1, e2, jnp.abs(e1 - e2), e1 * e2,
         gf1_ref[...], gf2_ref[...], fp1_ref[...], fp2_ref[...]], axis=1)
    h = jnp.dot(comb, wc1_ref[...], preferred_element_type=jnp.float32) + bc1_ref[...]
    h = jnp.maximum(_bn(h, gc1_ref[...], cc1_ref[...]), 0.0)
    h = jnp.dot(h, wc2_ref[...], preferred_element_type=jnp.float32) + bc2_ref[...]
    h = jnp.maximum(_bn(h, gc2_ref[...], cc2_ref[...]), 0.0)
    o_ref[...] = (
        jnp.dot(h, wc3_ref[...], preferred_element_type=jnp.float32) + bc3_ref[...]
    )


def _classifier(e1, e2, gf1, gf2, fp1, fp2, wc1, bc1, gc1, cc1,
                wc2, bc2, gc2, cc2, wc3, bc3):
    return pl.pallas_call(
        _classifier_body,
        out_shape=jax.ShapeDtypeStruct((NG, 1), jnp.float32),
    )(e1, e2, gf1, gf2, fp1, fp2,
      wc1, bc1.reshape(1, -1), gc1.reshape(1, -1), cc1.reshape(1, -1),
      wc2, bc2.reshape(1, -1), gc2.reshape(1, -1), cc2.reshape(1, -1),
      wc3, bc3.reshape(1, -1))


_SC_MESH = plsc.VectorSubcoreMesh(
    core_axis_name="c", subcore_axis_name="s", num_cores=NC, num_subcores=NS)


def _sc_edge_body(x_hbm, e_hbm, src_hbm, dst_hbm, z_hbm, out_hbm,
                  srcv0, dstv0, srcv1, dstv1, xv0, xv1, ev0, ev1,
                  agg, sg0, sg1, se0, se1, si0, si1):
    cid = lax.axis_index("c")
    sid = lax.axis_index("s")
    wid = cid * NS + sid
    rows_per_tile = _NROWS // NS

    pltpu.sync_copy(z_hbm, agg.at[pl.ds(sid * rows_per_tile, rows_per_tile)])
    plsc.subcore_barrier()

    tbase = wid * _EPB
    bufs = ((srcv0, dstv0, xv0, ev0, sg0, se0, si0),
            (srcv1, dstv1, xv1, ev1, sg1, se1, si1))

    def start_idx(c, b):
        srcv, dstv, si = bufs[b][0], bufs[b][1], bufs[b][6]
        base = tbase + c * _C
        pltpu.async_copy(src_hbm.at[pl.ds(base, _C)], srcv, si)
        pltpu.async_copy(dst_hbm.at[pl.ds(base, _C)], dstv, si)

    def wait_idx(b):
        srcv, dstv, si = bufs[b][0], bufs[b][1], bufs[b][6]
        pltpu.make_async_copy(src_hbm.at[pl.ds(0, _C)], srcv, si).wait()
        pltpu.make_async_copy(dst_hbm.at[pl.ds(0, _C)], dstv, si).wait()

    def start_fetch(c, b):
        srcv, xv, ev = bufs[b][0], bufs[b][2], bufs[b][3]
        pltpu.async_copy(x_hbm.at[srcv], xv, bufs[b][4])
        pltpu.async_copy(e_hbm.at[pl.ds(tbase + c * _C, _C)], ev, bufs[b][5])

    def finish(b, wait_sc):
        srcv, dstv, xv, ev = (bufs[b][0], bufs[b][1], bufs[b][2], bufs[b][3])
        pltpu.make_async_copy(x_hbm.at[srcv], xv, bufs[b][4]).wait()
        pltpu.make_async_copy(e_hbm.at[pl.ds(0, _C)], ev, bufs[b][5]).wait()

        def rows8(i, _):
            for rr in range(8):
                r = 8 * i + rr
                for j in range(F // 16):
                    sl = pl.ds(16 * j, 16)
                    xv[r, sl] = jnp.maximum(xv[r, sl] + ev[r, sl], 0.0)
            return 0

        lax.fori_loop(0, _C // 8, rows8, 0)

        pltpu.sync_copy(xv, agg.at[dstv], add=True)

    start_idx(0, 0)
    start_idx(1, 1)
    wait_idx(0)
    start_fetch(0, 0)
    wait_idx(1)
    start_fetch(1, 1)

    def step(s2, _):
        c0 = 2 * s2

        def half(b, c):
            has_next = c + 2 < _NCHUNK
            finish(b, s2 > 0)

            @pl.when(has_next)
            def _():
                start_idx(c + 2, b)
                wait_idx(b)
                start_fetch(c + 2, b)

        half(0, c0)
        half(1, c0 + 1)
        return 0

    lax.fori_loop(0, _NCHUNK // 2, step, 0)
    finish(0, True)

    plsc.subcore_barrier()
    pltpu.sync_copy(agg.at[pl.ds(sid * rows_per_tile, rows_per_tile)],
                    out_hbm.at[cid, pl.ds(sid * rows_per_tile, rows_per_tile)])


@functools.partial(
    pl.kernel,
    out_type=jax.ShapeDtypeStruct((NC, _NROWS, F), jnp.float32),
    mesh=_SC_MESH,
    scratch_types=(
        [pltpu.VMEM((_C,), jnp.int32)] * 4
        + [pltpu.VMEM((_C, F), jnp.float32)] * 4
        + [pltpu.VMEM_SHARED((_NROWS, F), jnp.float32)]
        + [pltpu.SemaphoreType.DMA] * 6
    ),
)
def _sc_edge(x_hbm, e_hbm, src_hbm, dst_hbm, z_hbm, out_hbm,
             srcv0, dstv0, srcv1, dstv1, xv0, xv1, ev0, ev1,
             agg, sg0, sg1, se0, se1, si0, si1):
    _sc_edge_body(x_hbm, e_hbm, src_hbm, dst_hbm, z_hbm, out_hbm,
                  srcv0, dstv0, srcv1, dstv1, xv0, xv1, ev0, ev1,
                  agg, sg0, sg1, se0, se1, si0, si1)


def _sc_maxpool_body(x_hbm, b_hbm, out_hbm, idv, xv, acc):
    cid = lax.axis_index("c")
    sid = lax.axis_index("s")
    wid = cid * NS + sid
    base = wid * _NPT

    neg = jnp.full((16,), -1e30, jnp.float32)

    def zrow(r, _):
        for j in range(EMB // 16):
            acc[r, pl.ds(16 * j, 16)] = neg
        return 0

    lax.fori_loop(0, _SEG, zrow, 0)

    pltpu.sync_copy(b_hbm.at[pl.ds(base, _NPT)], idv.at[pl.ds(0, _NPT)])
    pltpu.sync_copy(x_hbm.at[pl.ds(base, _NPT)], xv)

    def row(r, _):
        seg = idv[pl.ds(r, 16)][0]
        for j in range(EMB // 16):
            s = pl.ds(16 * j, 16)
            acc[seg, s] = jnp.maximum(acc[seg, s], xv[r, s])
        return 0

    lax.fori_loop(0, _NPT, row, 0)
    pltpu.sync_copy(acc, out_hbm.at[wid])


@functools.partial(
    pl.kernel,
    out_type=jax.ShapeDtypeStruct((NW, _SEG, EMB), jnp.float32),
    mesh=_SC_MESH,
    scratch_types=[
        pltpu.VMEM((_NPT + 16,), jnp.int32),
        pltpu.VMEM((_NPT, EMB), jnp.float32),
        pltpu.VMEM((_SEG, EMB), jnp.float32),
    ],
)
def _sc_maxpool(x_hbm, b_hbm, out_hbm, idv, xv, acc):
    _sc_maxpool_body(x_hbm, b_hbm, out_hbm, idv, xv, acc)


_LAYERS = [("l1_", "g1", "c1", True), ("l2_", "g2", "c2", True),
           ("l3_", "g3", "c3", False)]


def _pool(h, batch):
    xp = jnp.pad(h, ((0, _NPAD - N_NODES), (0, 0)))
    bp = jnp.pad(batch, (0, _NPAD - N_NODES), constant_values=NG)
    maxpart = _sc_maxpool(xp, bp)
    return _pool_combine(h, batch, maxpart)


def kernel(x1, edge_index1, edge_attr1, batch1, gf1, fp1,
           x2, edge_index2, edge_attr2, batch2, gf2, fp2,
           g0, c0, g1, c1, g2, c2, g3, c3,
           l1_We, l1_be, l1_W1, l1_b1, l1_g1, l1_c1, l1_W2, l1_b2,
           l2_We, l2_be, l2_W1, l2_b1, l2_g1, l2_c1, l2_W2, l2_b2,
           l3_We, l3_be, l3_W1, l3_b1, l3_g1, l3_c1, l3_W2, l3_b2,
           Wc1, bc1, gc1, cc1, Wc2, bc2, gc2, cc2, Wc3, bc3):
    p = dict(g0=g0, c0=c0, g1=g1, c1=c1, g2=g2, c2=c2, g3=g3, c3=c3)
    w = {}
    for pre, vals in [
        ("l1_", (l1_We, l1_be, l1_W1, l1_b1, l1_g1, l1_c1, l1_W2, l1_b2)),
        ("l2_", (l2_We, l2_be, l2_W1, l2_b1, l2_g1, l2_c1, l2_W2, l2_b2)),
        ("l3_", (l3_We, l3_be, l3_W1, l3_b1, l3_g1, l3_c1, l3_W2, l3_b2)),
    ]:
        for k, v in zip(("We", "be", "W1", "b1", "g1", "c1", "W2", "b2"), vals):
            w[pre + k] = v

    zeros = jnp.zeros((_NROWS // NS, F), jnp.float32)
    h = [_bn0(x1, g0, c0), _bn0(x2, g0, c0)]
    eis = [edge_index1, edge_index2]
    eas = [edge_attr1, edge_attr2]
    for pre, gg, cc, relu_out in _LAYERS:
        es = [_edge_linear(eas[m], w[pre + "We"], w[pre + "be"])
              for m in (0, 1)]
        aggs = [_sc_edge(h[m], es[m], eis[m][0], eis[m][1], zeros)
                for m in (0, 1)]
        h = [_node_mlp(h[m], aggs[m], w[pre + "W1"], w[pre + "b1"],
                       w[pre + "g1"], w[pre + "c1"], w[pre + "W2"],
                       w[pre + "b2"], p[gg], p[cc], relu_out)
             for m in (0, 1)]

    p1 = _pool(h[0], batch1)
    p2 = _pool(h[1], batch2)
    return _classifier(p1, p2, gf1, gf2, fp1, fp2,
                       Wc1, bc1, gc1, cc1, Wc2, bc2, gc2, cc2, Wc3, bc3)
